# baseline (speedup 1.0000x reference)
import functools

import jax
import jax.numpy as jnp
from jax import lax
from jax.experimental import pallas as pl
from jax.experimental.pallas import tpu as pltpu
from jax.experimental.pallas import tpu_sc as plsc

BATCH = 16384
FIELDS = 26
DIM = 32
FPAD = 32
NUM_EMB = 1000000

NC = 2
NS = 16
NW = NC * NS
B_PER_W = BATCH // NW
NBUF = 4
CHUNK_B = 32
NCHUNK = B_PER_W // CHUNK_B


REPACK_M = 5000


def _repack_tc(emb):
    def body(x_ref, o_ref):
        v = x_ref[...].reshape(REPACK_M, 4, DIM)
        parts = [v[:, k, :] for k in range(4)]
        o_ref[...] = lax.concatenate(parts, dimension=1)

    return pl.pallas_call(
        body,
        grid=(NUM_EMB // (4 * REPACK_M),),
        in_specs=[pl.BlockSpec((4 * REPACK_M, DIM), lambda i: (i, 0))],
        out_specs=pl.BlockSpec((REPACK_M, 128), lambda i: (i, 0)),
        out_shape=jax.ShapeDtypeStruct((NUM_EMB // 4, 128), jnp.float32),
    )(emb)


def kernel(x, embedding):
    idx = jnp.pad(x.astype(jnp.int32), ((0, 0), (0, FPAD - FIELDS)))
    idx = idx.reshape(BATCH * FPAD)
    emb_a = jax.lax.optimization_barrier(_repack_tc(embedding))
    emb_b = emb_a.reshape(NUM_EMB, DIM)
    mesh = plsc.VectorSubcoreMesh(core_axis_name="c", subcore_axis_name="s")

    @functools.partial(
        pl.kernel,
        mesh=mesh,
        out_type=jax.ShapeDtypeStruct((BATCH, FIELDS, DIM), jnp.float32),
        scratch_types=[
            pltpu.VMEM((B_PER_W * FPAD,), jnp.int32),
            [pltpu.VMEM((CHUNK_B, FIELDS, DIM), jnp.float32)
             for _ in range(NBUF)],
            [pltpu.SemaphoreType.DMA for _ in range(NBUF)],
            [pltpu.SemaphoreType.DMA for _ in range(NBUF)],
        ],
        compiler_params=pltpu.CompilerParams(use_tc_tiling_on_sc=False),
    )
    def gather_kernel(table_hbm, idx_hbm, out_hbm, idx_v, bufs, gsems, wsems):
        wid = lax.axis_index("s") * NC + lax.axis_index("c")
        base_b = wid * B_PER_W
        pltpu.sync_copy(idx_hbm.at[pl.ds(base_b * FPAD, B_PER_W * FPAD)],
                        idx_v)

        def start_gather(c, b):
            @pl.loop(0, CHUNK_B)
            def _(bi):
                pltpu.async_copy(
                    table_hbm.at[
                        idx_v.at[pl.ds((c * CHUNK_B + bi) * FPAD, FIELDS)]],
                    bufs[b].at[bi], gsems[b])

        def wait_gather(c, b):
            @pl.loop(0, CHUNK_B)
            def _(bi):
                pltpu.make_async_copy(
                    table_hbm.at[
                        idx_v.at[pl.ds((c * CHUNK_B + bi) * FPAD, FIELDS)]],
                    bufs[b].at[bi], gsems[b]).wait()

        def write(c, b):
            return pltpu.make_async_copy(
                bufs[b],
                out_hbm.at[pl.ds(base_b + c * CHUNK_B, CHUNK_B)], wsems[b])

        for b in range(NBUF):
            start_gather(b, b)

        @pl.loop(0, NCHUNK - NBUF, step=NBUF)
        def _(i):
            for b in range(NBUF):
                wait_gather(i + b, b)
                write(i + b, b).start()
            for b in range(NBUF):
                write(i + b, b).wait()
                start_gather(i + b + NBUF, b)

        for b in range(NBUF):
            wait_gather(NCHUNK - NBUF + b, b)
            write(NCHUNK - NBUF + b, b).start()
        for b in range(NBUF):
            write(NCHUNK - NBUF + b, b).wait()

    return gather_kernel(emb_b, idx)


# speedup vs baseline: 1.0006x; 1.0006x over previous
import functools

import jax
import jax.numpy as jnp
from jax import lax
from jax.experimental import pallas as pl
from jax.experimental.pallas import tpu as pltpu
from jax.experimental.pallas import tpu_sc as plsc

BATCH = 16384
FIELDS = 26
DIM = 32
FPAD = 32
NUM_EMB = 1000000

NC = 2
NS = 16
NW = NC * NS
B_PER_W = BATCH // NW
NBUF = 4
CHUNK_B = 32
NCHUNK = B_PER_W // CHUNK_B


REPACK_M = 5000


def _repack_tc(emb):
    def body(x_ref, o_ref):
        v = x_ref[...].reshape(REPACK_M, 4, DIM)
        parts = [v[:, k, :] for k in range(4)]
        o_ref[...] = lax.concatenate(parts, dimension=1)

    return pl.pallas_call(
        body,
        grid=(NUM_EMB // (4 * REPACK_M),),
        in_specs=[pl.BlockSpec((4 * REPACK_M, DIM), lambda i: (i, 0))],
        out_specs=pl.BlockSpec((REPACK_M, 128), lambda i: (i, 0)),
        out_shape=jax.ShapeDtypeStruct((NUM_EMB // 4, 128), jnp.float32),
    )(emb)


def kernel(x, embedding):
    idx = jnp.pad(x.astype(jnp.int32), ((0, 0), (0, FPAD - FIELDS)))
    idx = idx.reshape(BATCH * FPAD)
    emb_b = _repack_tc(embedding).reshape(NUM_EMB, DIM)
    mesh = plsc.VectorSubcoreMesh(core_axis_name="c", subcore_axis_name="s")

    @functools.partial(
        pl.kernel,
        mesh=mesh,
        out_type=jax.ShapeDtypeStruct((BATCH, FIELDS, DIM), jnp.float32),
        scratch_types=[
            pltpu.VMEM((B_PER_W * FPAD,), jnp.int32),
            [pltpu.VMEM((CHUNK_B, FIELDS, DIM), jnp.float32)
             for _ in range(NBUF)],
            [pltpu.SemaphoreType.DMA for _ in range(NBUF)],
            [pltpu.SemaphoreType.DMA for _ in range(NBUF)],
        ],
        compiler_params=pltpu.CompilerParams(use_tc_tiling_on_sc=False),
    )
    def gather_kernel(table_hbm, idx_hbm, out_hbm, idx_v, bufs, gsems, wsems):
        wid = lax.axis_index("s") * NC + lax.axis_index("c")
        base_b = wid * B_PER_W
        pltpu.sync_copy(idx_hbm.at[pl.ds(base_b * FPAD, B_PER_W * FPAD)],
                        idx_v)

        def start_gather(c, b):
            @pl.loop(0, CHUNK_B)
            def _(bi):
                pltpu.async_copy(
                    table_hbm.at[
                        idx_v.at[pl.ds((c * CHUNK_B + bi) * FPAD, FIELDS)]],
                    bufs[b].at[bi], gsems[b])

        def wait_gather(c, b):
            @pl.loop(0, CHUNK_B)
            def _(bi):
                pltpu.make_async_copy(
                    table_hbm.at[
                        idx_v.at[pl.ds((c * CHUNK_B + bi) * FPAD, FIELDS)]],
                    bufs[b].at[bi], gsems[b]).wait()

        def write(c, b):
            return pltpu.make_async_copy(
                bufs[b],
                out_hbm.at[pl.ds(base_b + c * CHUNK_B, CHUNK_B)], wsems[b])

        for b in range(NBUF):
            start_gather(b, b)

        @pl.loop(0, NCHUNK - NBUF, step=NBUF)
        def _(i):
            for b in range(NBUF):
                wait_gather(i + b, b)
                write(i + b, b).start()
            for b in range(NBUF):
                write(i + b, b).wait()
                start_gather(i + b + NBUF, b)

        for b in range(NBUF):
            wait_gather(NCHUNK - NBUF + b, b)
            write(NCHUNK - NBUF + b, b).start()
        for b in range(NBUF):
            write(NCHUNK - NBUF + b, b).wait()

    return gather_kernel(emb_b, idx)


# speedup vs baseline: 1.0910x; 1.0904x over previous
import functools

import jax
import jax.numpy as jnp
from jax import lax
from jax.experimental import pallas as pl
from jax.experimental.pallas import tpu as pltpu
from jax.experimental.pallas import tpu_sc as plsc

BATCH = 16384
FIELDS = 26
DIM = 32
FPAD = 32
NUM_EMB = 1000000

NC = 2
NS = 16
NW = NC * NS
B_PER_W = BATCH // NW
NBUF = 4
CHUNK_B = 32
NCHUNK = B_PER_W // CHUNK_B


def kernel(x, embedding):
    idx = jnp.pad(x.astype(jnp.int32), ((0, 0), (0, FPAD - FIELDS)))
    idx = idx.reshape(BATCH * FPAD)
    mesh = plsc.VectorSubcoreMesh(core_axis_name="c", subcore_axis_name="s")

    @functools.partial(
        pl.kernel,
        mesh=mesh,
        out_type=jax.ShapeDtypeStruct((BATCH * FIELDS, DIM), jnp.float32),
        scratch_types=[
            pltpu.VMEM((B_PER_W * FPAD,), jnp.int32),
            [pltpu.VMEM((CHUNK_B * FIELDS, DIM), jnp.float32)
             for _ in range(NBUF)],
            [pltpu.SemaphoreType.DMA for _ in range(NBUF)],
            [pltpu.SemaphoreType.DMA for _ in range(NBUF)],
        ],
        compiler_params=pltpu.CompilerParams(use_tc_tiling_on_sc=False),
    )
    def gather_kernel(table_hbm, idx_hbm, out_hbm, idx_v, bufs, gsems, wsems):
        wid = lax.axis_index("s") * NC + lax.axis_index("c")
        base_b = wid * B_PER_W
        pltpu.sync_copy(idx_hbm.at[pl.ds(base_b * FPAD, B_PER_W * FPAD)],
                        idx_v)

        def start_gather(c, b):
            @pl.loop(0, CHUNK_B)
            def _(bi):
                pltpu.async_copy(
                    table_hbm.at[
                        idx_v.at[pl.ds((c * CHUNK_B + bi) * FPAD, FIELDS)]],
                    bufs[b].at[pl.ds(bi * FIELDS, FIELDS)], gsems[b])

        def wait_gather(c, b):
            @pl.loop(0, CHUNK_B)
            def _(bi):
                pltpu.make_async_copy(
                    table_hbm.at[
                        idx_v.at[pl.ds((c * CHUNK_B + bi) * FPAD, FIELDS)]],
                    bufs[b].at[pl.ds(bi * FIELDS, FIELDS)],
                    gsems[b]).wait()

        def write(c, b):
            return pltpu.make_async_copy(
                bufs[b],
                out_hbm.at[pl.ds((base_b + c * CHUNK_B) * FIELDS,
                                 CHUNK_B * FIELDS)], wsems[b])

        for b in range(NBUF):
            start_gather(b, b)

        @pl.loop(0, NCHUNK - NBUF, step=NBUF)
        def _(i):
            for b in range(NBUF):
                wait_gather(i + b, b)
                write(i + b, b).start()
            for b in range(NBUF):
                write(i + b, b).wait()
                start_gather(i + b + NBUF, b)

        for b in range(NBUF):
            wait_gather(NCHUNK - NBUF + b, b)
            write(NCHUNK - NBUF + b, b).start()
        for b in range(NBUF):
            write(NCHUNK - NBUF + b, b).wait()

    out = gather_kernel(embedding, idx)
    out = jax.lax.optimization_barrier(out.reshape(BATCH * FIELDS * DIM // 128,
                                                   128))
    return out.reshape(BATCH, FIELDS, DIM)
